# baseline (speedup 1.0000x reference)
import functools

import jax
import jax.numpy as jnp
from jax import lax
from jax.experimental import pallas as pl
from jax.experimental.pallas import tpu as pltpu
from jax.experimental.pallas import tpu_sc as plsc

N, E, D, G, C = 10000, 320000, 128, 64, 2
NC, NS = 2, 16
NW = NC * NS
EPW = E // NW
CHUNK = 128
EPWP = 10240
NCHUNK = EPWP // CHUNK
NPAD = 10240
RPS = NPAD // NS
DEGW = 128
PAD_ROW = N

_MESH = dict(core_axis_name="c", subcore_axis_name="s", num_cores=NC,
             num_subcores=NS)


def _zero_fill(buf, rows, cols):
    @pl.loop(0, rows)
    def _(i):
        @pl.loop(0, cols, step=16)
        def _(j):
            buf[i, pl.ds(j, 16)] = jnp.zeros((16,), jnp.float32)


def _fill_ones(buf, rows, cols):
    @pl.loop(0, rows)
    def _(i):
        @pl.loop(0, cols, step=16)
        def _(j):
            buf[i, pl.ds(j, 16)] = jnp.ones((16,), jnp.float32)


def _sc_deg(rowp, interpret=False):
    mesh = plsc.VectorSubcoreMesh(**_MESH)

    @functools.partial(
        pl.kernel,
        out_type=jax.ShapeDtypeStruct((NC, NPAD, DEGW), jnp.float32),
        mesh=mesh,
        interpret=interpret,
        scratch_types=[
            pltpu.VMEM_SHARED((NPAD, DEGW), jnp.float32),
            pltpu.VMEM((NCHUNK, CHUNK), jnp.int32),
            pltpu.VMEM((CHUNK, DEGW), jnp.float32),
            pltpu.SemaphoreType.DMA,
        ],
    )
    def k(row_hbm, out_hbm, acc, row_v, ones_v, sem):
        c = lax.axis_index("c")
        s = lax.axis_index("s")
        wid = s * NC + c
        _zero_fill(ones_v, 80, DEGW)
        @pl.loop(0, RPS, step=80)
        def _(r):
            pltpu.sync_copy(ones_v.at[pl.ds(0, 80)],
                            acc.at[pl.ds(s * RPS + r, 80)])
        _fill_ones(ones_v, CHUNK, DEGW)
        pltpu.sync_copy(row_hbm.at[wid], row_v)
        plsc.subcore_barrier()
        @pl.loop(0, NCHUNK)
        def _(j):
            pltpu.async_copy(ones_v, acc.at[row_v.at[j]], sem, add=True)
        @pl.loop(0, NCHUNK)
        def _(j):
            pltpu.make_async_copy(ones_v, acc.at[row_v.at[j]], sem).wait()
        plsc.subcore_barrier()
        pltpu.sync_copy(acc.at[pl.ds(s * RPS, RPS)],
                        out_hbm.at[c].at[pl.ds(s * RPS, RPS)])

    return k(rowp)


def _sc_agg(u, rowp, colp, interpret=False):
    mesh = plsc.VectorSubcoreMesh(**_MESH)

    @functools.partial(
        pl.kernel,
        out_type=jax.ShapeDtypeStruct((NC, NPAD, D), jnp.float32),
        mesh=mesh,
        interpret=interpret,
        scratch_types=[
            pltpu.VMEM_SHARED((NPAD, D), jnp.float32),
            pltpu.VMEM((NCHUNK, CHUNK), jnp.int32),
            pltpu.VMEM((NCHUNK, CHUNK), jnp.int32),
            pltpu.VMEM((CHUNK, D), jnp.float32),
        ],
    )
    def k(u_hbm, row_hbm, col_hbm, out_hbm, acc, row_v, col_v, gbuf):
        c = lax.axis_index("c")
        s = lax.axis_index("s")
        wid = s * NC + c
        _zero_fill(gbuf, 80, D)
        @pl.loop(0, RPS, step=80)
        def _(r):
            pltpu.sync_copy(gbuf.at[pl.ds(0, 80)],
                            acc.at[pl.ds(s * RPS + r, 80)])
        pltpu.sync_copy(row_hbm.at[wid], row_v)
        pltpu.sync_copy(col_hbm.at[wid], col_v)
        plsc.subcore_barrier()
        @pl.loop(0, NCHUNK)
        def _(j):
            pltpu.sync_copy(u_hbm.at[col_v.at[j]], gbuf)
            pltpu.sync_copy(gbuf, acc.at[row_v.at[j]], add=True)
        plsc.subcore_barrier()
        pltpu.sync_copy(acc.at[pl.ds(s * RPS, RPS)],
                        out_hbm.at[c].at[pl.ds(s * RPS, RPS)])

    return k(u, rowp, colp)


BN = 1000


def _d_of(dp_ref):
    deg = dp_ref[0, :, 0] + dp_ref[1, :, 0]
    return jnp.where(deg > 0, lax.rsqrt(deg), 0.0)


def _tc_mm(x, wt, interpret=False):
    def body(x_ref, w_ref, o_ref):
        o_ref[...] = jnp.dot(x_ref[...], w_ref[...],
                             preferred_element_type=jnp.float32)

    return pl.pallas_call(
        body,
        grid=(N // BN,),
        in_specs=[
            pl.BlockSpec((BN, D), lambda i: (i, 0)),
            pl.BlockSpec((D, D), lambda i: (0, 0)),
        ],
        out_specs=pl.BlockSpec((BN, D), lambda i: (i, 0)),
        out_shape=jax.ShapeDtypeStruct((N, D), jnp.float32),
        interpret=interpret,
    )(x, wt)


def _tc_scale(y, degp, interpret=False):
    def body(y_ref, dp_ref, o_ref):
        d = _d_of(dp_ref)
        o_ref[...] = y_ref[...] * d[:, None]

    return pl.pallas_call(
        body,
        grid=(N // BN,),
        in_specs=[
            pl.BlockSpec((BN, D), lambda i: (i, 0)),
            pl.BlockSpec((NC, BN, DEGW), lambda i: (0, i, 0)),
        ],
        out_specs=pl.BlockSpec((BN, D), lambda i: (i, 0)),
        out_shape=jax.ShapeDtypeStruct((N, D), jnp.float32),
        interpret=interpret,
    )(y, degp)


def _tc_dense2(parts, degp, b1, w2t, interpret=False):
    def body(p_ref, dp_ref, b_ref, w_ref, o_ref):
        d = _d_of(dp_ref)
        agg = p_ref[0] + p_ref[1]
        h = jnp.maximum(agg * d[:, None] + b_ref[...], 0.0)
        o_ref[...] = jnp.dot(h, w_ref[...],
                             preferred_element_type=jnp.float32) * d[:, None]

    return pl.pallas_call(
        body,
        grid=(N // BN,),
        in_specs=[
            pl.BlockSpec((NC, BN, D), lambda i: (0, i, 0)),
            pl.BlockSpec((NC, BN, DEGW), lambda i: (0, i, 0)),
            pl.BlockSpec((1, D), lambda i: (0, 0)),
            pl.BlockSpec((D, D), lambda i: (0, 0)),
        ],
        out_specs=pl.BlockSpec((BN, D), lambda i: (i, 0)),
        out_shape=jax.ShapeDtypeStruct((N, D), jnp.float32),
        interpret=interpret,
    )(parts, degp, b1.reshape(1, D), w2t)


def _tc_final(parts, degp, b2, batch3, wlt, bl, interpret=False):
    def body(p_ref, dp_ref, b_ref, seg_ref, wl_ref, bl_ref, o_ref, pool_ref):
        i = pl.program_id(0)

        @pl.when(i == 0)
        def _():
            pool_ref[...] = jnp.zeros_like(pool_ref)

        d = _d_of(dp_ref)
        agg = p_ref[0] + p_ref[1]
        h = jnp.maximum(agg * d[:, None] + b_ref[...], 0.0)
        seg = seg_ref[0, 0, :]
        onehot = (seg[None, :] ==
                  lax.broadcasted_iota(jnp.int32, (G, BN), 0)).astype(jnp.float32)
        pool_ref[...] += jnp.dot(onehot, h, preferred_element_type=jnp.float32)

        @pl.when(i == pl.num_programs(0) - 1)
        def _():
            o_ref[...] = (jnp.dot(pool_ref[...], wl_ref[...],
                                  preferred_element_type=jnp.float32)
                          + bl_ref[...])

    return pl.pallas_call(
        body,
        grid=(N // BN,),
        in_specs=[
            pl.BlockSpec((NC, BN, D), lambda i: (0, i, 0)),
            pl.BlockSpec((NC, BN, DEGW), lambda i: (0, i, 0)),
            pl.BlockSpec((1, D), lambda i: (0, 0)),
            pl.BlockSpec((1, 1, BN), lambda i: (i, 0, 0)),
            pl.BlockSpec((D, C), lambda i: (0, 0)),
            pl.BlockSpec((1, C), lambda i: (0, 0)),
        ],
        out_specs=pl.BlockSpec((G, C), lambda i: (0, 0)),
        out_shape=jax.ShapeDtypeStruct((G, C), jnp.float32),
        scratch_shapes=[pltpu.VMEM((G, D), jnp.float32)],
        interpret=interpret,
    )(parts, degp, b2.reshape(1, D), batch3, wlt, bl.reshape(1, C))


def _pad_edges(e2, fill):
    pad = jnp.full((NW, EPWP - EPW), fill, dtype=e2.dtype)
    return jnp.concatenate([e2, pad], axis=1).reshape(NW, NCHUNK, CHUNK)


def _gnn(x, edge_index, batch, W1, b1, W2, b2, Wl, bl, interpret=False):
    rowp = _pad_edges(edge_index[0].reshape(NW, EPW), PAD_ROW)
    colp = _pad_edges(edge_index[1].reshape(NW, EPW), 0)
    batch3 = batch.reshape(N // BN, 1, BN)

    degp = _sc_deg(rowp, interpret=interpret)
    y1 = _tc_mm(x, W1.T, interpret=interpret)
    u1 = _tc_scale(y1, degp, interpret=interpret)
    parts1 = _sc_agg(u1, rowp, colp, interpret=interpret)
    u2 = _tc_dense2(parts1, degp, b1, W2.T, interpret=interpret)
    parts2 = _sc_agg(u2, rowp, colp, interpret=interpret)
    return _tc_final(parts2, degp, b2, batch3, Wl.T, bl, interpret=interpret)


def kernel(x, edge_index, batch, W1, b1, W2, b2, Wl, bl):
    return _gnn(x, edge_index, batch, W1, b1, W2, b2, Wl, bl)


# speedup vs baseline: 1.9300x; 1.9300x over previous
import functools

import jax
import jax.numpy as jnp
from jax import lax
from jax.experimental import pallas as pl
from jax.experimental.pallas import tpu as pltpu
from jax.experimental.pallas import tpu_sc as plsc

N, E, D, G, C = 10000, 320000, 128, 64, 2
NC, NS = 2, 16
NW = NC * NS
EPW = E // NW
CHUNK = 80
EPWP = 10000
NCHUNK = EPWP // CHUNK
NPAD = 10240
RPS = NPAD // NS
DEGW = 128
PAD_ROW = N

_MESH = dict(core_axis_name="c", subcore_axis_name="s", num_cores=NC,
             num_subcores=NS)


def _zero_fill(buf, rows, cols):
    @pl.loop(0, rows)
    def _(i):
        @pl.loop(0, cols, step=16)
        def _(j):
            buf[i, pl.ds(j, 16)] = jnp.zeros((16,), jnp.float32)


def _fill_ones(buf, rows, cols):
    @pl.loop(0, rows)
    def _(i):
        @pl.loop(0, cols, step=16)
        def _(j):
            buf[i, pl.ds(j, 16)] = jnp.ones((16,), jnp.float32)


def _sc_deg(rowp, interpret=False):
    mesh = plsc.VectorSubcoreMesh(**_MESH)

    @functools.partial(
        pl.kernel,
        out_type=jax.ShapeDtypeStruct((NC, NPAD, DEGW), jnp.float32),
        mesh=mesh,
        interpret=interpret,
        scratch_types=[
            pltpu.VMEM_SHARED((NPAD, DEGW), jnp.float32),
            pltpu.VMEM((NCHUNK, CHUNK), jnp.int32),
            pltpu.VMEM((CHUNK, DEGW), jnp.float32),
            pltpu.SemaphoreType.DMA,
        ],
    )
    def k(row_hbm, out_hbm, acc, row_v, ones_v, sem):
        c = lax.axis_index("c")
        s = lax.axis_index("s")
        wid = s * NC + c
        _zero_fill(ones_v, 80, DEGW)
        @pl.loop(0, RPS, step=80)
        def _(r):
            pltpu.sync_copy(ones_v.at[pl.ds(0, 80)],
                            acc.at[pl.ds(s * RPS + r, 80)])
        _fill_ones(ones_v, CHUNK, DEGW)
        pltpu.sync_copy(row_hbm.at[wid], row_v)
        plsc.subcore_barrier()
        @pl.loop(0, NCHUNK)
        def _(j):
            pltpu.async_copy(ones_v, acc.at[row_v.at[j]], sem, add=True)
        @pl.loop(0, NCHUNK)
        def _(j):
            pltpu.make_async_copy(ones_v, acc.at[row_v.at[j]], sem).wait()
        plsc.subcore_barrier()
        pltpu.sync_copy(acc.at[pl.ds(s * RPS, RPS)],
                        out_hbm.at[c].at[pl.ds(s * RPS, RPS)])

    return k(rowp)


def _sc_agg(u, rowp, colp, interpret=False):
    mesh = plsc.VectorSubcoreMesh(**_MESH)

    @functools.partial(
        pl.kernel,
        out_type=jax.ShapeDtypeStruct((NC, NPAD, D), jnp.float32),
        mesh=mesh,
        interpret=interpret,
        scratch_types=[
            pltpu.VMEM_SHARED((NPAD, D), jnp.float32),
            pltpu.VMEM((NCHUNK, CHUNK), jnp.int32),
            pltpu.VMEM((NCHUNK, CHUNK), jnp.int32),
            pltpu.VMEM((CHUNK, D), jnp.float32),
        ],
    )
    def k(u_hbm, row_hbm, col_hbm, out_hbm, acc, row_v, col_v, gbuf):
        c = lax.axis_index("c")
        s = lax.axis_index("s")
        wid = s * NC + c
        _zero_fill(gbuf, 80, D)
        @pl.loop(0, RPS, step=80)
        def _(r):
            pltpu.sync_copy(gbuf.at[pl.ds(0, 80)],
                            acc.at[pl.ds(s * RPS + r, 80)])
        pltpu.sync_copy(row_hbm.at[wid], row_v)
        pltpu.sync_copy(col_hbm.at[wid], col_v)
        plsc.subcore_barrier()
        @pl.loop(0, NCHUNK)
        def _(j):
            pltpu.sync_copy(u_hbm.at[col_v.at[j]], gbuf)
            pltpu.sync_copy(gbuf, acc.at[row_v.at[j]], add=True)
        plsc.subcore_barrier()
        pltpu.sync_copy(acc.at[pl.ds(s * RPS, RPS)],
                        out_hbm.at[c].at[pl.ds(s * RPS, RPS)])

    return k(u, rowp, colp)


BN = 1000


def _d_of(dp_ref):
    deg = dp_ref[0, :, 0] + dp_ref[1, :, 0]
    return jnp.where(deg > 0, lax.rsqrt(deg), 0.0)


def _tc_mm(x, wt, interpret=False):
    def body(x_ref, w_ref, o_ref):
        o_ref[...] = jnp.dot(x_ref[...], w_ref[...],
                             preferred_element_type=jnp.float32)

    return pl.pallas_call(
        body,
        grid=(N // BN,),
        in_specs=[
            pl.BlockSpec((BN, D), lambda i: (i, 0)),
            pl.BlockSpec((D, D), lambda i: (0, 0)),
        ],
        out_specs=pl.BlockSpec((BN, D), lambda i: (i, 0)),
        out_shape=jax.ShapeDtypeStruct((N, D), jnp.float32),
        interpret=interpret,
    )(x, wt)


def _tc_scale(y, degp, interpret=False):
    def body(y_ref, dp_ref, o_ref):
        d = _d_of(dp_ref)
        o_ref[...] = y_ref[...] * d[:, None]

    return pl.pallas_call(
        body,
        grid=(N // BN,),
        in_specs=[
            pl.BlockSpec((BN, D), lambda i: (i, 0)),
            pl.BlockSpec((NC, BN, DEGW), lambda i: (0, i, 0)),
        ],
        out_specs=pl.BlockSpec((BN, D), lambda i: (i, 0)),
        out_shape=jax.ShapeDtypeStruct((N, D), jnp.float32),
        interpret=interpret,
    )(y, degp)


def _tc_dense2(parts, degp, b1, w2t, interpret=False):
    def body(p_ref, dp_ref, b_ref, w_ref, o_ref):
        d = _d_of(dp_ref)
        agg = p_ref[0] + p_ref[1]
        h = jnp.maximum(agg * d[:, None] + b_ref[...], 0.0)
        o_ref[...] = jnp.dot(h, w_ref[...],
                             preferred_element_type=jnp.float32) * d[:, None]

    return pl.pallas_call(
        body,
        grid=(N // BN,),
        in_specs=[
            pl.BlockSpec((NC, BN, D), lambda i: (0, i, 0)),
            pl.BlockSpec((NC, BN, DEGW), lambda i: (0, i, 0)),
            pl.BlockSpec((1, D), lambda i: (0, 0)),
            pl.BlockSpec((D, D), lambda i: (0, 0)),
        ],
        out_specs=pl.BlockSpec((BN, D), lambda i: (i, 0)),
        out_shape=jax.ShapeDtypeStruct((N, D), jnp.float32),
        interpret=interpret,
    )(parts, degp, b1.reshape(1, D), w2t)


def _tc_final(parts, degp, b2, batch3, wlt, bl, interpret=False):
    def body(p_ref, dp_ref, b_ref, seg_ref, wl_ref, bl_ref, o_ref, pool_ref):
        i = pl.program_id(0)

        @pl.when(i == 0)
        def _():
            pool_ref[...] = jnp.zeros_like(pool_ref)

        d = _d_of(dp_ref)
        agg = p_ref[0] + p_ref[1]
        h = jnp.maximum(agg * d[:, None] + b_ref[...], 0.0)
        seg = seg_ref[0, 0, :]
        onehot = (seg[None, :] ==
                  lax.broadcasted_iota(jnp.int32, (G, BN), 0)).astype(jnp.float32)
        pool_ref[...] += jnp.dot(onehot, h, preferred_element_type=jnp.float32)

        @pl.when(i == pl.num_programs(0) - 1)
        def _():
            o_ref[...] = (jnp.dot(pool_ref[...], wl_ref[...],
                                  preferred_element_type=jnp.float32)
                          + bl_ref[...])

    return pl.pallas_call(
        body,
        grid=(N // BN,),
        in_specs=[
            pl.BlockSpec((NC, BN, D), lambda i: (0, i, 0)),
            pl.BlockSpec((NC, BN, DEGW), lambda i: (0, i, 0)),
            pl.BlockSpec((1, D), lambda i: (0, 0)),
            pl.BlockSpec((1, 1, BN), lambda i: (i, 0, 0)),
            pl.BlockSpec((D, C), lambda i: (0, 0)),
            pl.BlockSpec((1, C), lambda i: (0, 0)),
        ],
        out_specs=pl.BlockSpec((G, C), lambda i: (0, 0)),
        out_shape=jax.ShapeDtypeStruct((G, C), jnp.float32),
        scratch_shapes=[pltpu.VMEM((G, D), jnp.float32)],
        interpret=interpret,
    )(parts, degp, b2.reshape(1, D), batch3, wlt, bl.reshape(1, C))


def _pad_edges(e2, fill):
    pad = jnp.full((NW, EPWP - EPW), fill, dtype=e2.dtype)
    return jnp.concatenate([e2, pad], axis=1).reshape(NW, NCHUNK, CHUNK)


def _gnn(x, edge_index, batch, W1, b1, W2, b2, Wl, bl, interpret=False):
    rowp = _pad_edges(edge_index[0].reshape(NW, EPW), PAD_ROW)
    colp = _pad_edges(edge_index[1].reshape(NW, EPW), 0)
    batch3 = batch.reshape(N // BN, 1, BN)

    degp = _sc_deg(rowp, interpret=interpret)
    y1 = _tc_mm(x, W1.T, interpret=interpret)
    u1 = _tc_scale(y1, degp, interpret=interpret)
    parts1 = _sc_agg(u1, rowp, colp, interpret=interpret)
    u2 = _tc_dense2(parts1, degp, b1, W2.T, interpret=interpret)
    parts2 = _sc_agg(u2, rowp, colp, interpret=interpret)
    return _tc_final(parts2, degp, b2, batch3, Wl.T, bl, interpret=interpret)


def kernel(x, edge_index, batch, W1, b1, W2, b2, Wl, bl):
    return _gnn(x, edge_index, batch, W1, b1, W2, b2, Wl, bl)


# speedup vs baseline: 2.1616x; 1.1200x over previous
import functools

import jax
import jax.numpy as jnp
from jax import lax
from jax.experimental import pallas as pl
from jax.experimental.pallas import tpu as pltpu
from jax.experimental.pallas import tpu_sc as plsc

N, E, D, G, C = 10000, 320000, 128, 64, 2
NC, NS = 2, 16
NW = NC * NS
EPW = E // NW
CHUNK = 128
EPWP = 10240
NCHUNK = EPWP // CHUNK
NPAD = 10240
RPS = NPAD // NS
DEGW = 128
PAD_ROW = N

_MESH = dict(core_axis_name="c", subcore_axis_name="s", num_cores=NC,
             num_subcores=NS)


def _zero_fill(buf, rows, cols):
    @pl.loop(0, rows)
    def _(i):
        @pl.loop(0, cols, step=16)
        def _(j):
            buf[i, pl.ds(j, 16)] = jnp.zeros((16,), jnp.float32)


def _fill_ones(buf, rows, cols):
    @pl.loop(0, rows)
    def _(i):
        @pl.loop(0, cols, step=16)
        def _(j):
            buf[i, pl.ds(j, 16)] = jnp.ones((16,), jnp.float32)


def _sc_deg(rowp, interpret=False):
    mesh = plsc.VectorSubcoreMesh(**_MESH)

    @functools.partial(
        pl.kernel,
        out_type=jax.ShapeDtypeStruct((NC, NPAD, DEGW), jnp.float32),
        mesh=mesh,
        interpret=interpret,
        scratch_types=[
            pltpu.VMEM_SHARED((NPAD, DEGW), jnp.float32),
            pltpu.VMEM((NCHUNK, CHUNK), jnp.int32),
            pltpu.VMEM((CHUNK, DEGW), jnp.float32),
            pltpu.SemaphoreType.DMA,
        ],
    )
    def k(row_hbm, out_hbm, acc, row_v, ones_v, sem):
        c = lax.axis_index("c")
        s = lax.axis_index("s")
        wid = s * NC + c
        _zero_fill(ones_v, 80, DEGW)
        @pl.loop(0, RPS, step=80)
        def _(r):
            pltpu.sync_copy(ones_v.at[pl.ds(0, 80)],
                            acc.at[pl.ds(s * RPS + r, 80)])
        _fill_ones(ones_v, CHUNK, DEGW)
        pltpu.sync_copy(row_hbm.at[wid], row_v)
        plsc.subcore_barrier()
        @pl.loop(0, NCHUNK)
        def _(j):
            pltpu.async_copy(ones_v, acc.at[row_v.at[j]], sem, add=True)
        @pl.loop(0, NCHUNK)
        def _(j):
            pltpu.make_async_copy(ones_v, acc.at[row_v.at[j]], sem).wait()
        plsc.subcore_barrier()
        pltpu.sync_copy(acc.at[pl.ds(s * RPS, RPS)],
                        out_hbm.at[c].at[pl.ds(s * RPS, RPS)])

    return k(rowp)


def _sc_agg(u, rowp, colp, interpret=False):
    mesh = plsc.VectorSubcoreMesh(**_MESH)

    @functools.partial(
        pl.kernel,
        out_type=jax.ShapeDtypeStruct((NC, NPAD, D), jnp.float32),
        mesh=mesh,
        interpret=interpret,
        scratch_types=[
            pltpu.VMEM_SHARED((NPAD, D), jnp.float32),
            pltpu.VMEM((NCHUNK, CHUNK), jnp.int32),
            pltpu.VMEM((NCHUNK, CHUNK), jnp.int32),
            pltpu.VMEM((CHUNK, D), jnp.float32),
        ],
    )
    def k(u_hbm, row_hbm, col_hbm, out_hbm, acc, row_v, col_v, gbuf):
        c = lax.axis_index("c")
        s = lax.axis_index("s")
        wid = s * NC + c
        _zero_fill(gbuf, 80, D)
        @pl.loop(0, RPS, step=80)
        def _(r):
            pltpu.sync_copy(gbuf.at[pl.ds(0, 80)],
                            acc.at[pl.ds(s * RPS + r, 80)])
        pltpu.sync_copy(row_hbm.at[wid], row_v)
        pltpu.sync_copy(col_hbm.at[wid], col_v)
        plsc.subcore_barrier()
        @pl.loop(0, NCHUNK)
        def _(j):
            pltpu.sync_copy(u_hbm.at[col_v.at[j]], gbuf)
            pltpu.sync_copy(gbuf, acc.at[row_v.at[j]], add=True)
        plsc.subcore_barrier()
        pltpu.sync_copy(acc.at[pl.ds(s * RPS, RPS)],
                        out_hbm.at[c].at[pl.ds(s * RPS, RPS)])

    return k(u, rowp, colp)


BN = 1000


def _d_of(dp_ref):
    deg = dp_ref[0, :, 0] + dp_ref[1, :, 0]
    return jnp.where(deg > 0, lax.rsqrt(deg), 0.0)


def _tc_mm(x, wt, interpret=False):
    def body(x_ref, w_ref, o_ref):
        o_ref[...] = jnp.dot(x_ref[...], w_ref[...],
                             preferred_element_type=jnp.float32)

    return pl.pallas_call(
        body,
        grid=(N // BN,),
        in_specs=[
            pl.BlockSpec((BN, D), lambda i: (i, 0)),
            pl.BlockSpec((D, D), lambda i: (0, 0)),
        ],
        out_specs=pl.BlockSpec((BN, D), lambda i: (i, 0)),
        out_shape=jax.ShapeDtypeStruct((N, D), jnp.float32),
        interpret=interpret,
    )(x, wt)


def _tc_scale(y, degp, interpret=False):
    def body(y_ref, dp_ref, o_ref):
        d = _d_of(dp_ref)
        o_ref[...] = y_ref[...] * d[:, None]

    return pl.pallas_call(
        body,
        grid=(N // BN,),
        in_specs=[
            pl.BlockSpec((BN, D), lambda i: (i, 0)),
            pl.BlockSpec((NC, BN, DEGW), lambda i: (0, i, 0)),
        ],
        out_specs=pl.BlockSpec((BN, D), lambda i: (i, 0)),
        out_shape=jax.ShapeDtypeStruct((N, D), jnp.float32),
        interpret=interpret,
    )(y, degp)


def _tc_dense2(parts, degp, b1, w2t, interpret=False):
    def body(p_ref, dp_ref, b_ref, w_ref, o_ref):
        d = _d_of(dp_ref)
        agg = p_ref[0] + p_ref[1]
        h = jnp.maximum(agg * d[:, None] + b_ref[...], 0.0)
        o_ref[...] = jnp.dot(h, w_ref[...],
                             preferred_element_type=jnp.float32) * d[:, None]

    return pl.pallas_call(
        body,
        grid=(N // BN,),
        in_specs=[
            pl.BlockSpec((NC, BN, D), lambda i: (0, i, 0)),
            pl.BlockSpec((NC, BN, DEGW), lambda i: (0, i, 0)),
            pl.BlockSpec((1, D), lambda i: (0, 0)),
            pl.BlockSpec((D, D), lambda i: (0, 0)),
        ],
        out_specs=pl.BlockSpec((BN, D), lambda i: (i, 0)),
        out_shape=jax.ShapeDtypeStruct((N, D), jnp.float32),
        interpret=interpret,
    )(parts, degp, b1.reshape(1, D), w2t)


def _tc_final(parts, degp, b2, batch3, wlt, bl, interpret=False):
    def body(p_ref, dp_ref, b_ref, seg_ref, wl_ref, bl_ref, o_ref, pool_ref):
        i = pl.program_id(0)

        @pl.when(i == 0)
        def _():
            pool_ref[...] = jnp.zeros_like(pool_ref)

        d = _d_of(dp_ref)
        agg = p_ref[0] + p_ref[1]
        h = jnp.maximum(agg * d[:, None] + b_ref[...], 0.0)
        seg = seg_ref[0, 0, :]
        onehot = (seg[None, :] ==
                  lax.broadcasted_iota(jnp.int32, (G, BN), 0)).astype(jnp.float32)
        pool_ref[...] += jnp.dot(onehot, h, preferred_element_type=jnp.float32)

        @pl.when(i == pl.num_programs(0) - 1)
        def _():
            o_ref[...] = (jnp.dot(pool_ref[...], wl_ref[...],
                                  preferred_element_type=jnp.float32)
                          + bl_ref[...])

    return pl.pallas_call(
        body,
        grid=(N // BN,),
        in_specs=[
            pl.BlockSpec((NC, BN, D), lambda i: (0, i, 0)),
            pl.BlockSpec((NC, BN, DEGW), lambda i: (0, i, 0)),
            pl.BlockSpec((1, D), lambda i: (0, 0)),
            pl.BlockSpec((1, 1, BN), lambda i: (i, 0, 0)),
            pl.BlockSpec((D, C), lambda i: (0, 0)),
            pl.BlockSpec((1, C), lambda i: (0, 0)),
        ],
        out_specs=pl.BlockSpec((G, C), lambda i: (0, 0)),
        out_shape=jax.ShapeDtypeStruct((G, C), jnp.float32),
        scratch_shapes=[pltpu.VMEM((G, D), jnp.float32)],
        interpret=interpret,
    )(parts, degp, b2.reshape(1, D), batch3, wlt, bl.reshape(1, C))


def _pad_edges(e2, base):
    pad = jnp.broadcast_to(base + jnp.arange(EPWP - EPW, dtype=e2.dtype),
                           (NW, EPWP - EPW))
    return jnp.concatenate([e2, pad], axis=1).reshape(NW, NCHUNK, CHUNK)


def _gnn(x, edge_index, batch, W1, b1, W2, b2, Wl, bl, interpret=False):
    rowp = _pad_edges(edge_index[0].reshape(NW, EPW), PAD_ROW)
    colp = _pad_edges(edge_index[1].reshape(NW, EPW), 0)
    batch3 = batch.reshape(N // BN, 1, BN)

    degp = _sc_deg(rowp, interpret=interpret)
    y1 = _tc_mm(x, W1.T, interpret=interpret)
    u1 = _tc_scale(y1, degp, interpret=interpret)
    parts1 = _sc_agg(u1, rowp, colp, interpret=interpret)
    u2 = _tc_dense2(parts1, degp, b1, W2.T, interpret=interpret)
    parts2 = _sc_agg(u2, rowp, colp, interpret=interpret)
    return _tc_final(parts2, degp, b2, batch3, Wl.T, bl, interpret=interpret)


def kernel(x, edge_index, batch, W1, b1, W2, b2, Wl, bl):
    return _gnn(x, edge_index, batch, W1, b1, W2, b2, Wl, bl)


# speedup vs baseline: 2.4224x; 1.1206x over previous
import dataclasses
import functools

import jax
import jax.numpy as jnp
from jax import lax
from jax.experimental import pallas as pl
from jax.experimental.pallas import tpu as pltpu
from jax.experimental.pallas import tpu_sc as plsc

N, E, D, G, C = 10000, 320000, 128, 64, 2
NC, NS = 2, 16
NW = NC * NS
EPW = E // NW
CHUNK = 128
EPWP = 10240
NCHUNK = EPWP // CHUNK
NPAD = 10240
RPS = NPAD // NS
DEGW = 128
PAD_ROW = N

_MESH = dict(core_axis_name="c", subcore_axis_name="s", num_cores=NC,
             num_subcores=NS)


def _zero_fill(buf, rows, cols):
    @pl.loop(0, rows)
    def _(i):
        @pl.loop(0, cols, step=16)
        def _(j):
            buf[i, pl.ds(j, 16)] = jnp.zeros((16,), jnp.float32)


def _sc_deg(row2, interpret=False):
    mesh = plsc.VectorSubcoreMesh(**_MESH)
    cp = pltpu.CompilerParams()
    if "needs_layout_passes" in pltpu.CompilerParams.__dataclass_fields__:
        cp = dataclasses.replace(cp, needs_layout_passes=False)

    @functools.partial(
        pl.kernel,
        out_type=jax.ShapeDtypeStruct((NW, N), jnp.float32),
        mesh=mesh,
        interpret=interpret,
        compiler_params=cp,
        scratch_types=[
            pltpu.VMEM((N,), jnp.float32),
            pltpu.VMEM((EPW,), jnp.int32),
        ],
    )
    def k(row_hbm, out_hbm, dacc, row_v):
        c = lax.axis_index("c")
        s = lax.axis_index("s")
        wid = s * NC + c

        @pl.loop(0, N, step=16)
        def _(i):
            dacc[pl.ds(i, 16)] = jnp.zeros((16,), jnp.float32)

        pltpu.sync_copy(row_hbm.at[wid], row_v)
        ones16 = jnp.ones((16,), jnp.float32)

        @pl.loop(0, EPW, step=16)
        def _(e):
            idx = row_v[pl.ds(e, 16)]
            plsc.addupdate_scatter(dacc, [idx], ones16)

        pltpu.sync_copy(dacc, out_hbm.at[wid])

    return k(row2)


def _sc_agg(u, rowp, colp, interpret=False):
    mesh = plsc.VectorSubcoreMesh(**_MESH)

    @functools.partial(
        pl.kernel,
        out_type=jax.ShapeDtypeStruct((NC, NPAD, D), jnp.float32),
        mesh=mesh,
        interpret=interpret,
        scratch_types=[
            pltpu.VMEM_SHARED((NPAD, D), jnp.float32),
            pltpu.VMEM((NCHUNK, CHUNK), jnp.int32),
            pltpu.VMEM((NCHUNK, CHUNK), jnp.int32),
            pltpu.VMEM((CHUNK, D), jnp.float32),
        ],
    )
    def k(u_hbm, row_hbm, col_hbm, out_hbm, acc, row_v, col_v, gbuf):
        c = lax.axis_index("c")
        s = lax.axis_index("s")
        wid = s * NC + c
        _zero_fill(gbuf, 80, D)
        @pl.loop(0, RPS, step=80)
        def _(r):
            pltpu.sync_copy(gbuf.at[pl.ds(0, 80)],
                            acc.at[pl.ds(s * RPS + r, 80)])
        pltpu.sync_copy(row_hbm.at[wid], row_v)
        pltpu.sync_copy(col_hbm.at[wid], col_v)
        plsc.subcore_barrier()
        @pl.loop(0, NCHUNK)
        def _(j):
            pltpu.sync_copy(u_hbm.at[col_v.at[j]], gbuf)
            pltpu.sync_copy(gbuf, acc.at[row_v.at[j]], add=True)
        plsc.subcore_barrier()
        pltpu.sync_copy(acc.at[pl.ds(s * RPS, RPS)],
                        out_hbm.at[c].at[pl.ds(s * RPS, RPS)])

    return k(u, rowp, colp)


BN = 1000


def _d_of(dp_ref):
    deg = jnp.sum(dp_ref[0], axis=0)
    return jnp.where(deg > 0, lax.rsqrt(deg), 0.0)


def _tc_mm(x, wt, interpret=False):
    def body(x_ref, w_ref, o_ref):
        o_ref[...] = jnp.dot(x_ref[...], w_ref[...],
                             preferred_element_type=jnp.float32)

    return pl.pallas_call(
        body,
        grid=(N // BN,),
        in_specs=[
            pl.BlockSpec((BN, D), lambda i: (i, 0)),
            pl.BlockSpec((D, D), lambda i: (0, 0)),
        ],
        out_specs=pl.BlockSpec((BN, D), lambda i: (i, 0)),
        out_shape=jax.ShapeDtypeStruct((N, D), jnp.float32),
        interpret=interpret,
    )(x, wt)


def _tc_scale(y, degp, interpret=False):
    def body(y_ref, dp_ref, o_ref):
        d = _d_of(dp_ref)
        o_ref[...] = y_ref[...] * d[:, None]

    return pl.pallas_call(
        body,
        grid=(N // BN,),
        in_specs=[
            pl.BlockSpec((BN, D), lambda i: (i, 0)),
            pl.BlockSpec((1, NW, BN), lambda i: (i, 0, 0)),
        ],
        out_specs=pl.BlockSpec((BN, D), lambda i: (i, 0)),
        out_shape=jax.ShapeDtypeStruct((N, D), jnp.float32),
        interpret=interpret,
    )(y, degp)


def _tc_dense2(parts, degp, b1, w2t, interpret=False):
    def body(p_ref, dp_ref, b_ref, w_ref, o_ref):
        d = _d_of(dp_ref)
        agg = p_ref[0] + p_ref[1]
        h = jnp.maximum(agg * d[:, None] + b_ref[...], 0.0)
        o_ref[...] = jnp.dot(h, w_ref[...],
                             preferred_element_type=jnp.float32) * d[:, None]

    return pl.pallas_call(
        body,
        grid=(N // BN,),
        in_specs=[
            pl.BlockSpec((NC, BN, D), lambda i: (0, i, 0)),
            pl.BlockSpec((1, NW, BN), lambda i: (i, 0, 0)),
            pl.BlockSpec((1, D), lambda i: (0, 0)),
            pl.BlockSpec((D, D), lambda i: (0, 0)),
        ],
        out_specs=pl.BlockSpec((BN, D), lambda i: (i, 0)),
        out_shape=jax.ShapeDtypeStruct((N, D), jnp.float32),
        interpret=interpret,
    )(parts, degp, b1.reshape(1, D), w2t)


def _tc_final(parts, degp, b2, batch3, wlt, bl, interpret=False):
    def body(p_ref, dp_ref, b_ref, seg_ref, wl_ref, bl_ref, o_ref, pool_ref):
        i = pl.program_id(0)

        @pl.when(i == 0)
        def _():
            pool_ref[...] = jnp.zeros_like(pool_ref)

        d = _d_of(dp_ref)
        agg = p_ref[0] + p_ref[1]
        h = jnp.maximum(agg * d[:, None] + b_ref[...], 0.0)
        seg = seg_ref[0, 0, :]
        onehot = (seg[None, :] ==
                  lax.broadcasted_iota(jnp.int32, (G, BN), 0)).astype(jnp.float32)
        pool_ref[...] += jnp.dot(onehot, h, preferred_element_type=jnp.float32)

        @pl.when(i == pl.num_programs(0) - 1)
        def _():
            o_ref[...] = (jnp.dot(pool_ref[...], wl_ref[...],
                                  preferred_element_type=jnp.float32)
                          + bl_ref[...])

    return pl.pallas_call(
        body,
        grid=(N // BN,),
        in_specs=[
            pl.BlockSpec((NC, BN, D), lambda i: (0, i, 0)),
            pl.BlockSpec((1, NW, BN), lambda i: (i, 0, 0)),
            pl.BlockSpec((1, D), lambda i: (0, 0)),
            pl.BlockSpec((1, 1, BN), lambda i: (i, 0, 0)),
            pl.BlockSpec((D, C), lambda i: (0, 0)),
            pl.BlockSpec((1, C), lambda i: (0, 0)),
        ],
        out_specs=pl.BlockSpec((G, C), lambda i: (0, 0)),
        out_shape=jax.ShapeDtypeStruct((G, C), jnp.float32),
        scratch_shapes=[pltpu.VMEM((G, D), jnp.float32)],
        interpret=interpret,
    )(parts, degp, b2.reshape(1, D), batch3, wlt, bl.reshape(1, C))


def _pad_edges(e2, base):
    pad = jnp.broadcast_to(base + jnp.arange(EPWP - EPW, dtype=e2.dtype),
                           (NW, EPWP - EPW))
    return jnp.concatenate([e2, pad], axis=1).reshape(NW, NCHUNK, CHUNK)


def _gnn(x, edge_index, batch, W1, b1, W2, b2, Wl, bl, interpret=False):
    rowp = _pad_edges(edge_index[0].reshape(NW, EPW), PAD_ROW)
    colp = _pad_edges(edge_index[1].reshape(NW, EPW), 0)
    batch3 = batch.reshape(N // BN, 1, BN)

    degp = _sc_deg(edge_index[0].reshape(NW, EPW), interpret=interpret)
    degp = degp.reshape(NW, N // BN, BN).transpose(1, 0, 2)
    y1 = _tc_mm(x, W1.T, interpret=interpret)
    u1 = _tc_scale(y1, degp, interpret=interpret)
    parts1 = _sc_agg(u1, rowp, colp, interpret=interpret)
    u2 = _tc_dense2(parts1, degp, b1, W2.T, interpret=interpret)
    parts2 = _sc_agg(u2, rowp, colp, interpret=interpret)
    return _tc_final(parts2, degp, b2, batch3, Wl.T, bl, interpret=interpret)


def kernel(x, edge_index, batch, W1, b1, W2, b2, Wl, bl):
    return _gnn(x, edge_index, batch, W1, b1, W2, b2, Wl, bl)


# speedup vs baseline: 3.3643x; 1.3889x over previous
import dataclasses
import functools

import jax
import jax.numpy as jnp
from jax import lax
from jax.experimental import pallas as pl
from jax.experimental.pallas import tpu as pltpu
from jax.experimental.pallas import tpu_sc as plsc

N, E, D, G, C = 10000, 320000, 128, 64, 2
NC, NS = 2, 16
NW = NC * NS
EPW = E // NW
CHUNK = 80
EPWP = 10080
NCHUNK = EPWP // CHUNK
NPAD = 10240
RPS = NPAD // NS
DEGW = 128
PAD_ROW = N

_MESH = dict(core_axis_name="c", subcore_axis_name="s", num_cores=NC,
             num_subcores=NS)


def _zero_fill(buf, rows, cols):
    @pl.loop(0, rows)
    def _(i):
        @pl.loop(0, cols, step=16)
        def _(j):
            buf[i, pl.ds(j, 16)] = jnp.zeros((16,), jnp.float32)


def _sc_deg(row2, interpret=False):
    mesh = plsc.VectorSubcoreMesh(**_MESH)
    cp = pltpu.CompilerParams()
    if "needs_layout_passes" in pltpu.CompilerParams.__dataclass_fields__:
        cp = dataclasses.replace(cp, needs_layout_passes=False)

    @functools.partial(
        pl.kernel,
        out_type=jax.ShapeDtypeStruct((NW, N), jnp.float32),
        mesh=mesh,
        interpret=interpret,
        compiler_params=cp,
        scratch_types=[
            pltpu.VMEM((N,), jnp.float32),
            pltpu.VMEM((EPW,), jnp.int32),
        ],
    )
    def k(row_hbm, out_hbm, dacc, row_v):
        c = lax.axis_index("c")
        s = lax.axis_index("s")
        wid = s * NC + c

        @pl.loop(0, N, step=16)
        def _(i):
            dacc[pl.ds(i, 16)] = jnp.zeros((16,), jnp.float32)

        pltpu.sync_copy(row_hbm.at[wid], row_v)
        ones16 = jnp.ones((16,), jnp.float32)

        @pl.loop(0, EPW, step=16)
        def _(e):
            idx = row_v[pl.ds(e, 16)]
            plsc.addupdate_scatter(dacc, [idx], ones16)

        pltpu.sync_copy(dacc, out_hbm.at[wid])

    return k(row2)


def _sc_agg(u, rowp, colf, interpret=False):
    mesh = plsc.VectorSubcoreMesh(**_MESH)

    @functools.partial(
        pl.kernel,
        out_type=jax.ShapeDtypeStruct((NC, NPAD, D), jnp.float32),
        mesh=mesh,
        interpret=interpret,
        scratch_types=[
            pltpu.VMEM_SHARED((NPAD, D), jnp.float32),
            pltpu.VMEM((NCHUNK, CHUNK), jnp.int32),
            pltpu.VMEM((EPWP,), jnp.int32),
            pltpu.VMEM((CHUNK, D), jnp.float32),
            pltpu.VMEM((CHUNK, D), jnp.float32),
            pltpu.SemaphoreType.DMA,
            pltpu.SemaphoreType.DMA,
        ],
    )
    def k(u_hbm, row_hbm, col_hbm, out_hbm, acc, row_v, col_v, gb0, gb1,
          sem0, sem1):
        gbuf = (gb0, gb1)
        gsem = (sem0, sem1)
        c = lax.axis_index("c")
        s = lax.axis_index("s")
        wid = s * NC + c
        _zero_fill(gb0, 80, D)
        @pl.loop(0, RPS, step=80)
        def _(r):
            pltpu.sync_copy(gb0.at[pl.ds(0, 80)],
                            acc.at[pl.ds(s * RPS + r, 80)])
        pltpu.sync_copy(row_hbm.at[wid], row_v)
        pltpu.sync_copy(col_hbm.at[wid], col_v)
        plsc.subcore_barrier()

        def gidx(j):
            return col_v.at[pl.ds(j * CHUNK, CHUNK)]

        for b in range(2):
            pltpu.async_copy(u_hbm.at[gidx(b)], gbuf[b], gsem[b])

        @pl.loop(0, NCHUNK, step=2)
        def _(j0):
            for b in range(2):
                j = j0 + b
                pltpu.make_async_copy(u_hbm.at[gidx(j)], gbuf[b],
                                      gsem[b]).wait()
                pltpu.sync_copy(gbuf[b], acc.at[row_v.at[j]], add=True)
                @pl.when(j + 2 < NCHUNK)
                def _():
                    pltpu.async_copy(u_hbm.at[gidx(j + 2)], gbuf[b], gsem[b])
        plsc.subcore_barrier()
        pltpu.sync_copy(acc.at[pl.ds(s * RPS, RPS)],
                        out_hbm.at[c].at[pl.ds(s * RPS, RPS)])

    return k(u, rowp, colf)


BN = 1000


def _d_of(dp_ref):
    deg = jnp.sum(dp_ref[0], axis=0)
    return jnp.where(deg > 0, lax.rsqrt(deg), 0.0)


def _tc_mm_scale(x, wt, degp, interpret=False):
    def body(x_ref, w_ref, dp_ref, o_ref):
        d = _d_of(dp_ref)
        o_ref[...] = jnp.dot(x_ref[...], w_ref[...],
                             preferred_element_type=jnp.float32) * d[:, None]

    return pl.pallas_call(
        body,
        grid=(N // BN,),
        in_specs=[
            pl.BlockSpec((BN, D), lambda i: (i, 0)),
            pl.BlockSpec((D, D), lambda i: (0, 0)),
            pl.BlockSpec((1, NW, BN), lambda i: (i, 0, 0)),
        ],
        out_specs=pl.BlockSpec((BN, D), lambda i: (i, 0)),
        out_shape=jax.ShapeDtypeStruct((N, D), jnp.float32),
        interpret=interpret,
    )(x, wt, degp)


def _tc_dense2(parts, degp, b1, w2t, interpret=False):
    def body(p_ref, dp_ref, b_ref, w_ref, o_ref):
        d = _d_of(dp_ref)
        agg = p_ref[0] + p_ref[1]
        h = jnp.maximum(agg * d[:, None] + b_ref[...], 0.0)
        o_ref[...] = jnp.dot(h, w_ref[...],
                             preferred_element_type=jnp.float32) * d[:, None]

    return pl.pallas_call(
        body,
        grid=(N // BN,),
        in_specs=[
            pl.BlockSpec((NC, BN, D), lambda i: (0, i, 0)),
            pl.BlockSpec((1, NW, BN), lambda i: (i, 0, 0)),
            pl.BlockSpec((1, D), lambda i: (0, 0)),
            pl.BlockSpec((D, D), lambda i: (0, 0)),
        ],
        out_specs=pl.BlockSpec((BN, D), lambda i: (i, 0)),
        out_shape=jax.ShapeDtypeStruct((N, D), jnp.float32),
        interpret=interpret,
    )(parts, degp, b1.reshape(1, D), w2t)


def _tc_final(parts, degp, b2, batch3, wlt, bl, interpret=False):
    def body(p_ref, dp_ref, b_ref, seg_ref, wl_ref, bl_ref, o_ref, pool_ref):
        i = pl.program_id(0)

        @pl.when(i == 0)
        def _():
            pool_ref[...] = jnp.zeros_like(pool_ref)

        d = _d_of(dp_ref)
        agg = p_ref[0] + p_ref[1]
        h = jnp.maximum(agg * d[:, None] + b_ref[...], 0.0)
        seg = seg_ref[0, 0, :]
        onehot = (seg[None, :] ==
                  lax.broadcasted_iota(jnp.int32, (G, BN), 0)).astype(jnp.float32)
        pool_ref[...] += jnp.dot(onehot, h, preferred_element_type=jnp.float32)

        @pl.when(i == pl.num_programs(0) - 1)
        def _():
            o_ref[...] = (jnp.dot(pool_ref[...], wl_ref[...],
                                  preferred_element_type=jnp.float32)
                          + bl_ref[...])

    return pl.pallas_call(
        body,
        grid=(N // BN,),
        in_specs=[
            pl.BlockSpec((NC, BN, D), lambda i: (0, i, 0)),
            pl.BlockSpec((1, NW, BN), lambda i: (i, 0, 0)),
            pl.BlockSpec((1, D), lambda i: (0, 0)),
            pl.BlockSpec((1, 1, BN), lambda i: (i, 0, 0)),
            pl.BlockSpec((D, C), lambda i: (0, 0)),
            pl.BlockSpec((1, C), lambda i: (0, 0)),
        ],
        out_specs=pl.BlockSpec((G, C), lambda i: (0, 0)),
        out_shape=jax.ShapeDtypeStruct((G, C), jnp.float32),
        scratch_shapes=[pltpu.VMEM((G, D), jnp.float32)],
        interpret=interpret,
    )(parts, degp, b2.reshape(1, D), batch3, wlt, bl.reshape(1, C))


def _pad_flat(e2, base):
    pad = jnp.broadcast_to(base + jnp.arange(EPWP - EPW, dtype=e2.dtype),
                           (NW, EPWP - EPW))
    return jnp.concatenate([e2, pad], axis=1)


def _pad_edges(e2, base):
    pad = jnp.broadcast_to(base + jnp.arange(EPWP - EPW, dtype=e2.dtype),
                           (NW, EPWP - EPW))
    return jnp.concatenate([e2, pad], axis=1).reshape(NW, NCHUNK, CHUNK)


def _gnn(x, edge_index, batch, W1, b1, W2, b2, Wl, bl, interpret=False):
    rowp = _pad_edges(edge_index[0].reshape(NW, EPW), PAD_ROW)
    batch3 = batch.reshape(N // BN, 1, BN)

    degp = _sc_deg(edge_index[0].reshape(NW, EPW), interpret=interpret)
    degp = degp.reshape(NW, N // BN, BN).transpose(1, 0, 2)
    colf = _pad_flat(edge_index[1].reshape(NW, EPW), 0)
    u1 = _tc_mm_scale(x, W1.T, degp, interpret=interpret)
    parts1 = _sc_agg(u1, rowp, colf, interpret=interpret)
    u2 = _tc_dense2(parts1, degp, b1, W2.T, interpret=interpret)
    parts2 = _sc_agg(u2, rowp, colf, interpret=interpret)
    return _tc_final(parts2, degp, b2, batch3, Wl.T, bl, interpret=interpret)


def kernel(x, edge_index, batch, W1, b1, W2, b2, Wl, bl):
    return _gnn(x, edge_index, batch, W1, b1, W2, b2, Wl, bl)
